# baseline (speedup 1.0000x reference)
import jax
import jax.numpy as jnp
from jax.experimental import pallas as pl
from jax.experimental.pallas import tpu as pltpu

N = 100000
BN = 100352


def _tc_body(p_ref, q_ref, pred_ref, orig_ref, agg_ref, m_ref_arr,
             out_ref):
    vm = pred_ref[0, :]
    va = pred_ref[1, :]
    pd = orig_ref[2, :]
    qd = orig_ref[3, :]
    gs = orig_ref[4, :]
    bs = orig_ref[5, :]
    p = p_ref[...]
    q = q_ref[...]
    ag = agg_ref[...]
    m = m_ref_arr[...]
    m_pv = (m & 1) != 0
    m_ref = m >= 2
    vm2 = vm * vm
    qg = jnp.where(m != 0, q + qd - bs * vm2, 0.0)
    pg = jnp.where(m_ref, p + pd + gs * vm2, jnp.where(m_pv, ag, 0.0))
    out_ref[0, :] = vm
    out_ref[1, :] = va
    out_ref[2, :] = pg
    out_ref[3, :] = qg


def kernel(P_in, Q_in, bus_data_pred, bus_data_orig, agg_bus, mask_pv, mask_ref):
    mcomb = mask_pv.astype(jnp.int32) | (mask_ref.astype(jnp.int32) << 1)
    pred_t = bus_data_pred.T
    orig_t = bus_data_orig.T
    grid = (N + BN - 1) // BN
    out_t = pl.pallas_call(
        _tc_body,
        grid=(grid,),
        in_specs=[
            pl.BlockSpec((BN,), lambda j: (j,)),
            pl.BlockSpec((BN,), lambda j: (j,)),
            pl.BlockSpec((2, BN), lambda j: (0, j)),
            pl.BlockSpec((8, BN), lambda j: (0, j)),
            pl.BlockSpec((BN,), lambda j: (j,)),
            pl.BlockSpec((BN,), lambda j: (j,)),
        ],
        out_specs=pl.BlockSpec((4, BN), lambda j: (0, j)),
        out_shape=jax.ShapeDtypeStruct((4, N), jnp.float32),
    )(P_in, Q_in, pred_t, orig_t, agg_bus, mcomb)
    return out_t.T


# speedup vs baseline: 1.0973x; 1.0973x over previous
import jax
import jax.numpy as jnp
from jax.experimental import pallas as pl

N = 100000
BN = 50176


def _tc_body(p_ref, q_ref, pred_ref, orig_ref, agg_ref, m_ref_arr,
             out_ref):
    vm = pred_ref[0, :]
    va = pred_ref[1, :]
    pd = orig_ref[2, :]
    qd = orig_ref[3, :]
    gs = orig_ref[4, :]
    bs = orig_ref[5, :]
    p = p_ref[...]
    q = q_ref[...]
    ag = agg_ref[...]
    m = m_ref_arr[...]
    m_pv = (m & 1) != 0
    m_ref = m >= 2
    vm2 = vm * vm
    qg = jnp.where(m != 0, q + qd - bs * vm2, 0.0)
    pg = jnp.where(m_ref, p + pd + gs * vm2, jnp.where(m_pv, ag, 0.0))
    out_ref[0, :] = vm
    out_ref[1, :] = va
    out_ref[2, :] = pg
    out_ref[3, :] = qg


def kernel(P_in, Q_in, bus_data_pred, bus_data_orig, agg_bus, mask_pv, mask_ref):
    mcomb = mask_pv.astype(jnp.int32) | (mask_ref.astype(jnp.int32) << 1)
    pred_t = bus_data_pred.T
    orig_t = bus_data_orig.T
    grid = (N + BN - 1) // BN
    out_t = pl.pallas_call(
        _tc_body,
        grid=(grid,),
        in_specs=[
            pl.BlockSpec((BN,), lambda j: (j,)),
            pl.BlockSpec((BN,), lambda j: (j,)),
            pl.BlockSpec((2, BN), lambda j: (0, j)),
            pl.BlockSpec((8, BN), lambda j: (0, j)),
            pl.BlockSpec((BN,), lambda j: (j,)),
            pl.BlockSpec((BN,), lambda j: (j,)),
        ],
        out_specs=pl.BlockSpec((4, BN), lambda j: (0, j)),
        out_shape=jax.ShapeDtypeStruct((4, N), jnp.float32),
    )(P_in, Q_in, pred_t, orig_t, agg_bus, mcomb)
    return out_t.T
